# baseline (speedup 1.0000x reference)
import functools

import jax
import jax.numpy as jnp
from jax import lax
from jax.experimental import pallas as pl
from jax.experimental.pallas import tpu as pltpu
from jax.experimental.pallas import tpu_sc as plsc

N = 2900000
GAMMA = 0.8
RHO = 8.0
EPS = 1e-14
B, D = 256, 256

_NC, _NS = 1, 16
_NW = _NC * _NS
_CW = B // _NW
_NT = 6


def _gather_body(tau_i_hbm, b_i_hbm, s_i_hbm, tau_t_hbm, b_t_hbm, s_t_hbm,
                 img_ids_hbm, txt_ids_hbm, blk,
                 idx_v, val_v, sem):
    wid = lax.axis_index("s") * _NC + lax.axis_index("c")
    base = pl.multiple_of(wid * _CW, _CW)

    tabs = (tau_i_hbm, b_i_hbm, s_i_hbm, tau_t_hbm, b_t_hbm, s_t_hbm)

    copies = [pltpu.async_copy(ids.at[pl.ds(base, _CW)], idx_v.at[j], sem)
              for j, ids in enumerate((img_ids_hbm, txt_ids_hbm))]
    for cp in copies:
        cp.wait()
    copies = [pltpu.async_copy(tabs[t].at[idx_v.at[t // 3]], val_v.at[t], sem)
              for t in range(_NT)]
    for cp in copies:
        cp.wait()
    copies = [pltpu.async_copy(val_v.at[t], blk.at[t, pl.ds(base, _CW)], sem)
              for t in range(_NT)]
    for cp in copies:
        cp.wait()
    copies = [pltpu.async_copy(val_v.at[t], blk.at[t, pl.ds(base, _CW)], sem)
              for t in range(_NT)]
    for cp in copies:
        cp.wait()
    copies = [pltpu.async_copy(val_v.at[t], blk.at[t, pl.ds(base, _CW)], sem)
              for t in range(_NT)]
    for cp in copies:
        cp.wait()


@functools.cache
def _make_gather_call():
    return functools.partial(
        pl.kernel,
        out_type=jax.ShapeDtypeStruct((8, B), jnp.float32),
        mesh=plsc.VectorSubcoreMesh(core_axis_name="c", subcore_axis_name="s",
                                    num_cores=_NC, num_subcores=_NS),
        scratch_types=[
            pltpu.VMEM((2, _CW), jnp.int32),
            pltpu.VMEM((_NT, _CW), jnp.float32),
            pltpu.SemaphoreType.DMA,
        ],
    )(_gather_body)


def _matmul_body(img_ref, txt_ref, x_ref, y_ref, diag_ref):
    x = lax.dot_general(img_ref[...], txt_ref[...],
                        (((1,), (1,)), ((), ())),
                        preferred_element_type=jnp.float32)
    x_ref[...] = x
    y_ref[...] = lax.dot_general(txt_ref[...], img_ref[...],
                                 (((1,), (1,)), ((), ())),
                                 preferred_element_type=jnp.float32)
    r = lax.broadcasted_iota(jnp.int32, (B, B), 0)
    c = lax.broadcasted_iota(jnp.int32, (B, B), 1)
    diag_ref[...] = jnp.sum(jnp.where(r == c, x, jnp.zeros_like(x)),
                            axis=0, keepdims=True)


_matmul_call = pl.pallas_call(
    _matmul_body,
    out_shape=[
        jax.ShapeDtypeStruct((B, B), jnp.float32),
        jax.ShapeDtypeStruct((B, B), jnp.float32),
        jax.ShapeDtypeStruct((1, B), jnp.float32),
    ],
)


def _dense_body(x_ref, y_ref, diag_ref, blk_ref, epoch_ref, loss_ref,
                tau_img_ref, tau_txt_ref):
    x = x_ref[...]
    y = y_ref[...]
    r = lax.broadcasted_iota(jnp.int32, (B, B), 0)
    c = lax.broadcasted_iota(jnp.int32, (B, B), 1)
    on_diag = r == c
    zero = jnp.zeros_like(x)
    diag_row = diag_ref[...]

    isf = jnp.where(epoch_ref[0] == 0, 1.0, 0.0)

    def side(m, tau_row, b_row, s_row):
        diffs = m - diag_row
        dt = diffs * (1.0 / tau_row)
        b_new = jnp.maximum(jnp.max(dt, axis=0, keepdims=True), b_row)
        e = jnp.where(on_diag, zero, jnp.exp(dt - b_new))
        g = jnp.sum(e, axis=0, keepdims=True)
        s_upd = (1.0 - GAMMA) * s_row * jnp.exp(b_row - b_new) + GAMMA * g
        s_v = isf * g + (1.0 - isf) * s_upd
        p = jnp.sum(e * diffs, axis=0, keepdims=True)
        return jnp.sum(p * (1.0 / jnp.maximum(s_v, EPS)))

    img_loss = side(y, blk_ref[0:1, :], blk_ref[1:2, :], blk_ref[2:3, :])
    txt_loss = side(x, blk_ref[3:4, :], blk_ref[4:5, :], blk_ref[5:6, :])
    loss_ref[0, 0] = img_loss / B + txt_loss / B
    tau_img_ref[...] = lax.squeeze(blk_ref[0:1, :], (0,))
    tau_txt_ref[...] = lax.squeeze(blk_ref[3:4, :], (0,))


_dense_call = pl.pallas_call(
    _dense_body,
    in_specs=[
        pl.BlockSpec(),
        pl.BlockSpec(),
        pl.BlockSpec(),
        pl.BlockSpec(),
        pl.BlockSpec(memory_space=pltpu.SMEM),
    ],
    out_shape=[
        jax.ShapeDtypeStruct((1, 1), jnp.float32),
        jax.ShapeDtypeStruct((B,), jnp.float32),
        jax.ShapeDtypeStruct((B,), jnp.float32),
    ],
    out_specs=[
        pl.BlockSpec(memory_space=pltpu.SMEM),
        pl.BlockSpec(),
        pl.BlockSpec(),
    ],
)


def kernel(image_features, text_features, image_ids, text_ids, epoch,
           max_epoch, s_I, s_T, b_I, b_T, tau_I, tau_T, u_I, u_T):
    x, y, diag_row = _matmul_call(image_features, text_features)
    blk = _make_gather_call()(
        tau_I, b_I, s_I, tau_T, b_T, s_T, image_ids, text_ids)

    epoch_arr = jnp.asarray(epoch, jnp.int32).reshape(1)
    loss, tau_img, tau_txt = _dense_call(x, y, diag_row, blk, epoch_arr)

    return (loss.reshape(()), tau_img, tau_txt)


# speedup vs baseline: 1.0155x; 1.0155x over previous
import functools

import jax
import jax.numpy as jnp
from jax import lax
from jax.experimental import pallas as pl
from jax.experimental.pallas import tpu as pltpu
from jax.experimental.pallas import tpu_sc as plsc

N = 2900000
GAMMA = 0.8
RHO = 8.0
EPS = 1e-14
B, D = 256, 256

_NC, _NS = 1, 16
_NW = _NC * _NS
_CW = B // _NW
_NT = 6


def _gather_body(tau_i_hbm, b_i_hbm, s_i_hbm, tau_t_hbm, b_t_hbm, s_t_hbm,
                 img_ids_hbm, txt_ids_hbm, blk,
                 idx_v, val_v, sem):
    wid = lax.axis_index("s") * _NC + lax.axis_index("c")
    base = pl.multiple_of(wid * _CW, _CW)

    tabs = (tau_i_hbm, b_i_hbm, s_i_hbm, tau_t_hbm, b_t_hbm, s_t_hbm)

    copies = [pltpu.async_copy(ids.at[pl.ds(base, _CW)], idx_v.at[j], sem)
              for j, ids in enumerate((img_ids_hbm, txt_ids_hbm))]
    for cp in copies:
        cp.wait()
    copies = [pltpu.async_copy(tabs[t].at[idx_v.at[t // 3]], val_v.at[t], sem)
              for t in range(_NT)]
    for cp in copies:
        cp.wait()
    copies = [pltpu.async_copy(val_v.at[t], blk.at[t, pl.ds(base, _CW)], sem)
              for t in range(_NT)]
    for cp in copies:
        cp.wait()


@functools.cache
def _make_gather_call():
    return functools.partial(
        pl.kernel,
        out_type=jax.ShapeDtypeStruct((8, B), jnp.float32),
        mesh=plsc.VectorSubcoreMesh(core_axis_name="c", subcore_axis_name="s",
                                    num_cores=_NC, num_subcores=_NS),
        scratch_types=[
            pltpu.VMEM((2, _CW), jnp.int32),
            pltpu.VMEM((_NT, _CW), jnp.float32),
            pltpu.SemaphoreType.DMA,
        ],
    )(_gather_body)


def _matmul_body(img_ref, txt_ref, dx_ref, dy_ref, mx_ref, my_ref):
    x = lax.dot_general(img_ref[...], txt_ref[...],
                        (((1,), (1,)), ((), ())),
                        preferred_element_type=jnp.float32)
    y = lax.dot_general(txt_ref[...], img_ref[...],
                        (((1,), (1,)), ((), ())),
                        preferred_element_type=jnp.float32)
    r = lax.broadcasted_iota(jnp.int32, (B, B), 0)
    c = lax.broadcasted_iota(jnp.int32, (B, B), 1)
    diag_row = jnp.sum(jnp.where(r == c, x, jnp.zeros_like(x)),
                       axis=0, keepdims=True)
    dx = x - diag_row
    dy = y - diag_row
    dx_ref[...] = dx
    dy_ref[...] = dy
    mx_ref[...] = jnp.max(dx, axis=0, keepdims=True)
    my_ref[...] = jnp.max(dy, axis=0, keepdims=True)


_matmul_call = pl.pallas_call(
    _matmul_body,
    out_shape=[
        jax.ShapeDtypeStruct((B, B), jnp.float32),
        jax.ShapeDtypeStruct((B, B), jnp.float32),
        jax.ShapeDtypeStruct((1, B), jnp.float32),
        jax.ShapeDtypeStruct((1, B), jnp.float32),
    ],
)


def _dense_body(dx_ref, dy_ref, mx_ref, my_ref, blk_ref, epoch_ref, loss_ref,
                tau_img_ref, tau_txt_ref):
    r = lax.broadcasted_iota(jnp.int32, (B, B), 0)
    c = lax.broadcasted_iota(jnp.int32, (B, B), 1)
    on_diag = r == c
    zero = jnp.zeros((B, B), jnp.float32)

    isf = jnp.where(epoch_ref[0] == 0, 1.0, 0.0)

    def side(diffs, dmax, tau_row, b_row, s_row):
        rcp_tau = 1.0 / tau_row
        dt = diffs * rcp_tau
        b_new = jnp.maximum(dmax * rcp_tau, b_row)
        e = jnp.where(on_diag, zero, jnp.exp(dt - b_new))
        g = jnp.sum(e, axis=0, keepdims=True)
        s_upd = (1.0 - GAMMA) * s_row * jnp.exp(b_row - b_new) + GAMMA * g
        s_v = isf * g + (1.0 - isf) * s_upd
        p = jnp.sum(e * diffs, axis=0, keepdims=True)
        return jnp.sum(p * (1.0 / jnp.maximum(s_v, EPS)))

    img_loss = side(dy_ref[...], my_ref[...],
                    blk_ref[0:1, :], blk_ref[1:2, :], blk_ref[2:3, :])
    txt_loss = side(dx_ref[...], mx_ref[...],
                    blk_ref[3:4, :], blk_ref[4:5, :], blk_ref[5:6, :])
    loss_ref[0, 0] = img_loss / B + txt_loss / B
    tau_img_ref[...] = lax.squeeze(blk_ref[0:1, :], (0,))
    tau_txt_ref[...] = lax.squeeze(blk_ref[3:4, :], (0,))


_dense_call = pl.pallas_call(
    _dense_body,
    in_specs=[
        pl.BlockSpec(),
        pl.BlockSpec(),
        pl.BlockSpec(),
        pl.BlockSpec(),
        pl.BlockSpec(),
        pl.BlockSpec(memory_space=pltpu.SMEM),
    ],
    out_shape=[
        jax.ShapeDtypeStruct((1, 1), jnp.float32),
        jax.ShapeDtypeStruct((B,), jnp.float32),
        jax.ShapeDtypeStruct((B,), jnp.float32),
    ],
    out_specs=[
        pl.BlockSpec(memory_space=pltpu.SMEM),
        pl.BlockSpec(),
        pl.BlockSpec(),
    ],
)


def kernel(image_features, text_features, image_ids, text_ids, epoch,
           max_epoch, s_I, s_T, b_I, b_T, tau_I, tau_T, u_I, u_T):
    dx, dy, mx, my = _matmul_call(image_features, text_features)
    blk = _make_gather_call()(
        tau_I, b_I, s_I, tau_T, b_T, s_T, image_ids, text_ids)

    epoch_arr = jnp.asarray(epoch, jnp.int32).reshape(1)
    loss, tau_img, tau_txt = _dense_call(dx, dy, mx, my, blk, epoch_arr)

    return (loss.reshape(()), tau_img, tau_txt)
